# baseline (speedup 1.0000x reference)
import functools

import jax
import jax.numpy as jnp
from jax import lax
from jax.experimental import pallas as pl
from jax.experimental.pallas import tpu as pltpu
from jax.experimental.pallas import tpu_sc as plsc

NUM_CARDS = 100000
NUM_ACTIONS = 8
BATCH = 16384

L = 16
NC, NS = 2, 16
NW = NC * NS
BPW = BATCH // NW
CHUNK = 128
NCHUNK = BPW // CHUNK

_mesh = plsc.VectorSubcoreMesh(core_axis_name="c", subcore_axis_name="s")


@functools.partial(
    pl.kernel,
    mesh=_mesh,
    compiler_params=pltpu.CompilerParams(use_tc_tiling_on_sc=False),
    out_type=(
        jax.ShapeDtypeStruct((BATCH, NUM_ACTIONS), jnp.float32),
        jax.ShapeDtypeStruct((BATCH, NUM_ACTIONS), jnp.float32),
    ),
    scratch_types=[
        pltpu.VMEM((BPW,), jnp.int32),
        pltpu.VMEM((BPW,), jnp.int32),
        pltpu.VMEM((BPW,), jnp.int32),
        pltpu.VMEM((BPW,), jnp.int32),
        pltpu.VMEM((BPW,), jnp.int32),
        pltpu.VMEM((BPW, NUM_ACTIONS), jnp.float32),
        pltpu.VMEM((BPW, NUM_ACTIONS), jnp.float32),
        pltpu.SemaphoreType.DMA,
        pltpu.SemaphoreType.DMA,
    ],
)
def _qnet_sc(c0_hbm, c1_hbm, u0_hbm, ug_hbm, w0_hbm, w1_hbm,
             q0_hbm, q1_hbm,
             idx0_v, c1_v, u0_v, ug_v, idx1_v, rows0_v, rows1_v,
             sem0, sem1):
    wid = lax.axis_index("s") * NC + lax.axis_index("c")
    base = wid * BPW

    pltpu.sync_copy(c0_hbm.at[pl.ds(base, BPW)], idx0_v)
    pltpu.sync_copy(c1_hbm.at[pl.ds(base, BPW)], c1_v)
    pltpu.sync_copy(u0_hbm.at[pl.ds(base, BPW)], u0_v)
    pltpu.sync_copy(ug_hbm.at[pl.ds(base, BPW)], ug_v)

    gathers0 = [
        pltpu.async_copy(
            w0_hbm.at[idx0_v.at[pl.ds(j * CHUNK, CHUNK)]],
            rows0_v.at[pl.ds(j * CHUNK, CHUNK)],
            sem0,
        )
        for j in range(NCHUNK)
    ]

    for i in range(BPW // L):
        sl = pl.ds(i * L, L)
        idx1_v[sl] = (c1_v[sl] * (NUM_ACTIONS * NUM_ACTIONS)
                      + u0_v[sl] * NUM_ACTIONS + ug_v[sl])

    gathers1 = [
        pltpu.async_copy(
            w1_hbm.at[idx1_v.at[pl.ds(j * CHUNK, CHUNK)]],
            rows1_v.at[pl.ds(j * CHUNK, CHUNK)],
            sem1,
        )
        for j in range(NCHUNK)
    ]

    for g in gathers0:
        g.wait()
    pltpu.sync_copy(rows0_v, q0_hbm.at[pl.ds(base, BPW)])
    for g in gathers1:
        g.wait()
    pltpu.sync_copy(rows1_v, q1_hbm.at[pl.ds(base, BPW)])


def kernel(cards_0, cards_1, u0, u0_greedy, weights_0, weights_1):
    return _qnet_sc(
        cards_0.astype(jnp.int32),
        cards_1.astype(jnp.int32),
        u0.astype(jnp.int32),
        u0_greedy.astype(jnp.int32),
        weights_0,
        weights_1,
    )


# speedup vs baseline: 75.9832x; 75.9832x over previous
import functools

import jax
import jax.numpy as jnp
from jax import lax
from jax.experimental import pallas as pl
from jax.experimental.pallas import tpu as pltpu
from jax.experimental.pallas import tpu_sc as plsc

NUM_CARDS = 100000
NUM_ACTIONS = 8
BATCH = 16384

L = 16
NC, NS = 2, 16
NW = NC * NS
BPW = BATCH // NW
NG = BPW // L
NIDX = BPW * NUM_ACTIONS
CHUNK = 128
NCHUNK = NIDX // CHUNK

W0_PAD_ROWS = 100096
W0_TILES = W0_PAD_ROWS // 128
W1_TILES = NUM_CARDS * NUM_ACTIONS ** 2 // 128

_mesh = plsc.VectorSubcoreMesh(core_axis_name="c", subcore_axis_name="s")


@functools.partial(
    pl.kernel,
    mesh=_mesh,
    compiler_params=pltpu.CompilerParams(use_tc_tiling_on_sc=False),
    out_type=(
        jax.ShapeDtypeStruct((BATCH * NUM_ACTIONS,), jnp.float32),
        jax.ShapeDtypeStruct((BATCH * NUM_ACTIONS,), jnp.float32),
    ),
    scratch_types=[
        pltpu.VMEM((BPW,), jnp.int32),
        pltpu.VMEM((BPW,), jnp.int32),
        pltpu.VMEM((BPW,), jnp.int32),
        pltpu.VMEM((BPW,), jnp.int32),
        pltpu.VMEM((NIDX,), jnp.int32),
        pltpu.VMEM((NIDX,), jnp.int32),
        pltpu.VMEM((NIDX,), jnp.float32),
        pltpu.VMEM((NIDX,), jnp.float32),
        pltpu.SemaphoreType.DMA,
        pltpu.SemaphoreType.DMA,
        pltpu.SemaphoreType.DMA,
    ],
)
def _qnet_sc(c0_hbm, c1_hbm, u0_hbm, ug_hbm, w0_hbm, w1_hbm,
             q0_hbm, q1_hbm,
             c0_v, c1_v, u0_v, ug_v, gix0_v, gix1_v, out0_v, out1_v,
             sem_in, sem0, sem1):
    wid = lax.axis_index("s") * NC + lax.axis_index("c")
    base = wid * BPW

    in_copies = [
        pltpu.async_copy(c0_hbm.at[pl.ds(base, BPW)], c0_v, sem_in),
        pltpu.async_copy(c1_hbm.at[pl.ds(base, BPW)], c1_v, sem_in),
        pltpu.async_copy(u0_hbm.at[pl.ds(base, BPW)], u0_v, sem_in),
        pltpu.async_copy(ug_hbm.at[pl.ds(base, BPW)], ug_v, sem_in),
    ]
    for cp in in_copies:
        cp.wait()

    def build(gix_v, c_of_i):
        def body(i, carry):
            c = c_of_i(i)
            off = ((c >> 7) << 10) + (c & 127)
            pos = (i >> 3) * 1024 + (i & 7) * L
            for a in range(NUM_ACTIONS):
                gix_v[pl.ds(pos + a * 128, L)] = off + a * 128
            return carry
        lax.fori_loop(0, NG, body, 0)

    def fire(w_hbm, gix_v, out_v, sem):
        return [
            pltpu.async_copy(
                w_hbm.at[gix_v.at[pl.ds(k * CHUNK, CHUNK)]],
                out_v.at[pl.ds(k * CHUNK, CHUNK)],
                sem,
            )
            for k in range(NCHUNK)
        ]

    build(gix0_v, lambda i: c0_v[pl.ds(i * L, L)])
    copies0 = fire(w0_hbm, gix0_v, out0_v, sem0)

    def joint(i):
        sl = pl.ds(i * L, L)
        return (c1_v[sl] * (NUM_ACTIONS * NUM_ACTIONS)
                + u0_v[sl] * NUM_ACTIONS + ug_v[sl])
    build(gix1_v, joint)
    copies1 = fire(w1_hbm, gix1_v, out1_v, sem1)

    for cp in copies0:
        cp.wait()
    pltpu.sync_copy(out0_v, q0_hbm.at[pl.ds(wid * NIDX, NIDX)])
    for cp in copies1:
        cp.wait()
    pltpu.sync_copy(out1_v, q1_hbm.at[pl.ds(wid * NIDX, NIDX)])


def _flat_view(w, tiles):
    return (w.reshape(tiles, 128, NUM_ACTIONS)
            .transpose(0, 2, 1)
            .reshape(tiles * 1024))


def _untile_out(qf):
    return (qf.reshape(BATCH // 128, NUM_ACTIONS, 128)
            .transpose(0, 2, 1)
            .reshape(BATCH, NUM_ACTIONS))


def kernel(cards_0, cards_1, u0, u0_greedy, weights_0, weights_1):
    w0v = _flat_view(
        jnp.pad(weights_0, ((0, W0_PAD_ROWS - NUM_CARDS), (0, 0))), W0_TILES)
    w1v = _flat_view(weights_1, W1_TILES)
    q0f, q1f = _qnet_sc(
        cards_0.astype(jnp.int32),
        cards_1.astype(jnp.int32),
        u0.astype(jnp.int32),
        u0_greedy.astype(jnp.int32),
        w0v,
        w1v,
    )
    return (_untile_out(q0f), _untile_out(q1f))


# speedup vs baseline: 77.5742x; 1.0209x over previous
import functools

import jax
import jax.numpy as jnp
from jax import lax
from jax.experimental import pallas as pl
from jax.experimental.pallas import tpu as pltpu
from jax.experimental.pallas import tpu_sc as plsc

NUM_CARDS = 100000
NUM_ACTIONS = 8
BATCH = 16384

L = 16
NC, NS = 2, 16
NW = NC * NS
BPW = BATCH // NW
NG = BPW // L
NIDX = BPW * NUM_ACTIONS
CHUNK = 128
NCHUNK = NIDX // CHUNK

W0_PAD_ROWS = 100096
W0_TILES = W0_PAD_ROWS // 128
W1_TILES = NUM_CARDS * NUM_ACTIONS ** 2 // 128

_mesh = plsc.VectorSubcoreMesh(core_axis_name="c", subcore_axis_name="s")


@functools.partial(
    pl.kernel,
    mesh=_mesh,
    compiler_params=pltpu.CompilerParams(use_tc_tiling_on_sc=False),
    out_type=(
        jax.ShapeDtypeStruct((BATCH * NUM_ACTIONS,), jnp.float32),
        jax.ShapeDtypeStruct((BATCH * NUM_ACTIONS,), jnp.float32),
    ),
    scratch_types=[
        pltpu.VMEM((BPW,), jnp.int32),
        pltpu.VMEM((BPW,), jnp.int32),
        pltpu.VMEM((BPW,), jnp.int32),
        pltpu.VMEM((BPW,), jnp.int32),
        pltpu.VMEM((NIDX,), jnp.int32),
        pltpu.VMEM((NIDX,), jnp.int32),
        pltpu.VMEM((NIDX,), jnp.float32),
        pltpu.VMEM((NIDX,), jnp.float32),
        pltpu.SemaphoreType.DMA,
        pltpu.SemaphoreType.DMA,
        pltpu.SemaphoreType.DMA,
    ],
)
def _qnet_sc(c0_hbm, c1_hbm, u0_hbm, ug_hbm, w0_hbm, w1_hbm,
             q0_hbm, q1_hbm,
             c0_v, c1_v, u0_v, ug_v, gix0_v, gix1_v, out0_v, out1_v,
             sem_in, sem0, sem1):
    wid = lax.axis_index("s") * NC + lax.axis_index("c")
    base = wid * BPW

    in_copies = [
        pltpu.async_copy(c0_hbm.at[pl.ds(base, BPW)], c0_v, sem_in),
        pltpu.async_copy(c1_hbm.at[pl.ds(base, BPW)], c1_v, sem_in),
        pltpu.async_copy(u0_hbm.at[pl.ds(base, BPW)], u0_v, sem_in),
        pltpu.async_copy(ug_hbm.at[pl.ds(base, BPW)], ug_v, sem_in),
    ]
    for cp in in_copies:
        cp.wait()

    def build(gix_v, c_of_i):
        def body(i, carry):
            c = c_of_i(i)
            off = ((c >> 7) << 10) + (c & 127)
            pos = (i >> 3) * 1024 + (i & 7) * L
            for a in range(NUM_ACTIONS):
                gix_v[pl.ds(pos + a * 128, L)] = off + a * 128
            return carry
        lax.fori_loop(0, NG, body, 0)

    def fire(w_hbm, gix_v, out_v, sem):
        return [pltpu.async_copy(w_hbm.at[gix_v], out_v, sem)]

    build(gix0_v, lambda i: c0_v[pl.ds(i * L, L)])
    copies0 = fire(w0_hbm, gix0_v, out0_v, sem0)

    def joint(i):
        sl = pl.ds(i * L, L)
        return (c1_v[sl] * (NUM_ACTIONS * NUM_ACTIONS)
                + u0_v[sl] * NUM_ACTIONS + ug_v[sl])
    build(gix1_v, joint)
    copies1 = fire(w1_hbm, gix1_v, out1_v, sem1)

    for cp in copies0:
        cp.wait()
    wb0 = pltpu.async_copy(out0_v, q0_hbm.at[pl.ds(wid * NIDX, NIDX)], sem_in)
    for cp in copies1:
        cp.wait()
    wb1 = pltpu.async_copy(out1_v, q1_hbm.at[pl.ds(wid * NIDX, NIDX)], sem_in)
    wb0.wait()
    wb1.wait()


def _flat_view(w, tiles):
    return (w.reshape(tiles, 128, NUM_ACTIONS)
            .transpose(0, 2, 1)
            .reshape(tiles * 1024))


def _untile_out(qf):
    return (qf.reshape(BATCH // 128, NUM_ACTIONS, 128)
            .transpose(0, 2, 1)
            .reshape(BATCH, NUM_ACTIONS))


def kernel(cards_0, cards_1, u0, u0_greedy, weights_0, weights_1):
    w0v = _flat_view(
        jnp.pad(weights_0, ((0, W0_PAD_ROWS - NUM_CARDS), (0, 0))), W0_TILES)
    w1v = _flat_view(weights_1, W1_TILES)
    q0f, q1f = _qnet_sc(
        cards_0.astype(jnp.int32),
        cards_1.astype(jnp.int32),
        u0.astype(jnp.int32),
        u0_greedy.astype(jnp.int32),
        w0v,
        w1v,
    )
    return (_untile_out(q0f), _untile_out(q1f))


# speedup vs baseline: 77.7027x; 1.0017x over previous
import functools

import jax
import jax.numpy as jnp
from jax import lax
from jax.experimental import pallas as pl
from jax.experimental.pallas import tpu as pltpu
from jax.experimental.pallas import tpu_sc as plsc

NUM_CARDS = 100000
NUM_ACTIONS = 8
BATCH = 16384

L = 16
NC, NS = 2, 16
NW = NC * NS
BPW = BATCH // NW
NG = BPW // L
NIDX = BPW * NUM_ACTIONS
CHUNK = 128
NCHUNK = NIDX // CHUNK

W0_PAD_ROWS = 100096
W0_TILES = W0_PAD_ROWS // 128
W1_TILES = NUM_CARDS * NUM_ACTIONS ** 2 // 128

_mesh = plsc.VectorSubcoreMesh(core_axis_name="c", subcore_axis_name="s")


@functools.partial(
    pl.kernel,
    mesh=_mesh,
    compiler_params=pltpu.CompilerParams(use_tc_tiling_on_sc=False),
    out_type=(
        jax.ShapeDtypeStruct((BATCH * NUM_ACTIONS,), jnp.float32),
        jax.ShapeDtypeStruct((BATCH * NUM_ACTIONS,), jnp.float32),
    ),
    scratch_types=[
        pltpu.VMEM((BPW,), jnp.int32),
        pltpu.VMEM((BPW,), jnp.int32),
        pltpu.VMEM((BPW,), jnp.int32),
        pltpu.VMEM((BPW,), jnp.int32),
        pltpu.VMEM((NIDX,), jnp.int32),
        pltpu.VMEM((NIDX,), jnp.int32),
        pltpu.VMEM((NIDX,), jnp.float32),
        pltpu.VMEM((NIDX,), jnp.float32),
        pltpu.SemaphoreType.DMA,
        pltpu.SemaphoreType.DMA,
        pltpu.SemaphoreType.DMA,
    ],
)
def _qnet_sc(c0_hbm, c1_hbm, u0_hbm, ug_hbm, w0_hbm, w1_hbm,
             q0_hbm, q1_hbm,
             c0_v, c1_v, u0_v, ug_v, gix0_v, gix1_v, out0_v, out1_v,
             sem_in, sem0, sem1):
    wid = lax.axis_index("s") * NC + lax.axis_index("c")
    base = wid * BPW

    cp_c0 = pltpu.async_copy(c0_hbm.at[pl.ds(base, BPW)], c0_v, sem0)
    in_copies = [
        pltpu.async_copy(c1_hbm.at[pl.ds(base, BPW)], c1_v, sem_in),
        pltpu.async_copy(u0_hbm.at[pl.ds(base, BPW)], u0_v, sem_in),
        pltpu.async_copy(ug_hbm.at[pl.ds(base, BPW)], ug_v, sem_in),
    ]
    cp_c0.wait()

    def build(gix_v, c_of_i, g0, g1):
        def body(i, carry):
            c = c_of_i(i)
            off = ((c >> 7) << 10) + (c & 127)
            pos = (i >> 3) * 1024 + (i & 7) * L
            for a in range(NUM_ACTIONS):
                gix_v[pl.ds(pos + a * 128, L)] = off + a * 128
            return carry
        lax.fori_loop(g0, g1, body, 0)

    def fire(w_hbm, gix_v, out_v, sem, k0, k1):
        return pltpu.async_copy(
            w_hbm.at[gix_v.at[pl.ds(k0 * 1024, (k1 - k0) * 1024)]],
            out_v.at[pl.ds(k0 * 1024, (k1 - k0) * 1024)],
            sem,
        )

    c0_of = lambda i: c0_v[pl.ds(i * L, L)]
    build(gix0_v, c0_of, 0, NG // 2)
    copies0 = [fire(w0_hbm, gix0_v, out0_v, sem0, 0, 2)]
    build(gix0_v, c0_of, NG // 2, NG)
    copies0.append(fire(w0_hbm, gix0_v, out0_v, sem0, 2, 4))

    for cp in in_copies:
        cp.wait()

    def joint(i):
        sl = pl.ds(i * L, L)
        return (c1_v[sl] * (NUM_ACTIONS * NUM_ACTIONS)
                + u0_v[sl] * NUM_ACTIONS + ug_v[sl])
    build(gix1_v, joint, 0, NG)
    copies1 = [fire(w1_hbm, gix1_v, out1_v, sem1, 0, 4)]

    for cp in copies0:
        cp.wait()
    wb0 = pltpu.async_copy(out0_v, q0_hbm.at[pl.ds(wid * NIDX, NIDX)], sem_in)
    for cp in copies1:
        cp.wait()
    wb1 = pltpu.async_copy(out1_v, q1_hbm.at[pl.ds(wid * NIDX, NIDX)], sem_in)
    wb0.wait()
    wb1.wait()


def _flat_view(w, tiles):
    return (w.reshape(tiles, 128, NUM_ACTIONS)
            .transpose(0, 2, 1)
            .reshape(tiles * 1024))


def _untile_out(qf):
    return (qf.reshape(BATCH // 128, NUM_ACTIONS, 128)
            .transpose(0, 2, 1)
            .reshape(BATCH, NUM_ACTIONS))


def kernel(cards_0, cards_1, u0, u0_greedy, weights_0, weights_1):
    w0v = _flat_view(
        jnp.pad(weights_0, ((0, W0_PAD_ROWS - NUM_CARDS), (0, 0))), W0_TILES)
    w1v = _flat_view(weights_1, W1_TILES)
    q0f, q1f = _qnet_sc(
        cards_0.astype(jnp.int32),
        cards_1.astype(jnp.int32),
        u0.astype(jnp.int32),
        u0_greedy.astype(jnp.int32),
        w0v,
        w1v,
    )
    return (_untile_out(q0f), _untile_out(q1f))


# speedup vs baseline: 78.1507x; 1.0058x over previous
import functools

import jax
import jax.numpy as jnp
from jax import lax
from jax.experimental import pallas as pl
from jax.experimental.pallas import tpu as pltpu
from jax.experimental.pallas import tpu_sc as plsc

NUM_CARDS = 100000
NUM_ACTIONS = 8
BATCH = 16384

L = 16
NC, NS = 2, 16
NW = NC * NS
BPW = BATCH // NW
NG = BPW // L
NIDX = BPW * NUM_ACTIONS
CHUNK = 128
NCHUNK = NIDX // CHUNK

W0_PAD_ROWS = 100096
W0_TILES = W0_PAD_ROWS // 128
W1_TILES = NUM_CARDS * NUM_ACTIONS ** 2 // 128

_mesh = plsc.VectorSubcoreMesh(core_axis_name="c", subcore_axis_name="s")


@functools.partial(
    pl.kernel,
    mesh=_mesh,
    compiler_params=pltpu.CompilerParams(use_tc_tiling_on_sc=False),
    out_type=(
        jax.ShapeDtypeStruct((BATCH * NUM_ACTIONS,), jnp.float32),
        jax.ShapeDtypeStruct((BATCH * NUM_ACTIONS,), jnp.float32),
    ),
    scratch_types=[
        pltpu.VMEM((BPW,), jnp.int32),
        pltpu.VMEM((BPW,), jnp.int32),
        pltpu.VMEM((BPW,), jnp.int32),
        pltpu.VMEM((BPW,), jnp.int32),
        pltpu.VMEM((NIDX,), jnp.int32),
        pltpu.VMEM((NIDX,), jnp.int32),
        pltpu.VMEM((NIDX,), jnp.float32),
        pltpu.VMEM((NIDX,), jnp.float32),
        pltpu.SemaphoreType.DMA,
        pltpu.SemaphoreType.DMA,
        pltpu.SemaphoreType.DMA,
        pltpu.SemaphoreType.DMA,
        pltpu.SemaphoreType.DMA,
    ],
)
def _qnet_sc(c0_hbm, c1_hbm, u0_hbm, ug_hbm, w0_hbm, w1_hbm,
             q0_hbm, q1_hbm,
             c0_v, c1_v, u0_v, ug_v, gix0_v, gix1_v, out0_v, out1_v,
             sem_in, sem0, sem1, sem2, sem3):
    wid = lax.axis_index("s") * NC + lax.axis_index("c")
    base = wid * BPW

    cp_c0 = pltpu.async_copy(c0_hbm.at[pl.ds(base, BPW)], c0_v, sem0)
    in_copies = [
        pltpu.async_copy(c1_hbm.at[pl.ds(base, BPW)], c1_v, sem_in),
        pltpu.async_copy(u0_hbm.at[pl.ds(base, BPW)], u0_v, sem_in),
        pltpu.async_copy(ug_hbm.at[pl.ds(base, BPW)], ug_v, sem_in),
    ]
    cp_c0.wait()

    def build(gix_v, c_of_i, g0, g1):
        def body(i, carry):
            c = c_of_i(i)
            off = ((c >> 7) << 10) + (c & 127)
            pos = (i >> 3) * 1024 + (i & 7) * L
            for a in range(NUM_ACTIONS):
                gix_v[pl.ds(pos + a * 128, L)] = off + a * 128
            return carry
        lax.fori_loop(g0, g1, body, 0)

    def fire(w_hbm, gix_v, out_v, sem, k0, k1):
        return pltpu.async_copy(
            w_hbm.at[gix_v.at[pl.ds(k0 * 1024, (k1 - k0) * 1024)]],
            out_v.at[pl.ds(k0 * 1024, (k1 - k0) * 1024)],
            sem,
        )

    c0_of = lambda i: c0_v[pl.ds(i * L, L)]
    build(gix0_v, c0_of, 0, NG // 2)
    g0a = fire(w0_hbm, gix0_v, out0_v, sem0, 0, 2)
    build(gix0_v, c0_of, NG // 2, NG)
    g0b = fire(w0_hbm, gix0_v, out0_v, sem2, 2, 4)

    for cp in in_copies:
        cp.wait()

    def joint(i):
        sl = pl.ds(i * L, L)
        return (c1_v[sl] * (NUM_ACTIONS * NUM_ACTIONS)
                + u0_v[sl] * NUM_ACTIONS + ug_v[sl])
    build(gix1_v, joint, 0, NG // 2)
    g1a = fire(w1_hbm, gix1_v, out1_v, sem1, 0, 2)
    build(gix1_v, joint, NG // 2, NG)
    g1b = fire(w1_hbm, gix1_v, out1_v, sem3, 2, 4)

    HALF = NIDX // 2
    wbs = []
    for g, out_v_, q_hbm_, h in ((g0a, out0_v, q0_hbm, 0),
                                 (g0b, out0_v, q0_hbm, 1),
                                 (g1a, out1_v, q1_hbm, 0),
                                 (g1b, out1_v, q1_hbm, 1)):
        g.wait()
        wbs.append(pltpu.async_copy(
            out_v_.at[pl.ds(h * HALF, HALF)],
            q_hbm_.at[pl.ds(wid * NIDX + h * HALF, HALF)],
            sem_in,
        ))
    for wb in wbs:
        wb.wait()


def _flat_view(w, tiles):
    return (w.reshape(tiles, 128, NUM_ACTIONS)
            .transpose(0, 2, 1)
            .reshape(tiles * 1024))


def _untile_out(qf):
    return (qf.reshape(BATCH // 128, NUM_ACTIONS, 128)
            .transpose(0, 2, 1)
            .reshape(BATCH, NUM_ACTIONS))


def kernel(cards_0, cards_1, u0, u0_greedy, weights_0, weights_1):
    w0v = _flat_view(
        jnp.pad(weights_0, ((0, W0_PAD_ROWS - NUM_CARDS), (0, 0))), W0_TILES)
    w1v = _flat_view(weights_1, W1_TILES)
    q0f, q1f = _qnet_sc(
        cards_0.astype(jnp.int32),
        cards_1.astype(jnp.int32),
        u0.astype(jnp.int32),
        u0_greedy.astype(jnp.int32),
        w0v,
        w1v,
    )
    return (_untile_out(q0f), _untile_out(q1f))
